# baseline (speedup 1.0000x reference)
import functools

import jax
import jax.numpy as jnp
from jax import lax
from jax.experimental import pallas as pl
from jax.experimental.pallas import tpu as pltpu
from jax.experimental.pallas import tpu_sc as plsc

_B = 16384
_H = 50
_D = 32
_V = 1000000

_NW = 32
_TB = _V // 128
_TAIL = _V - _TB * 128
_NBLK = (_TB + _NW - 1) // _NW

_BPW = _B // _NW


def _make_transpose():
  mesh = plsc.VectorSubcoreMesh(core_axis_name="c", subcore_axis_name="s")

  @functools.partial(
      pl.kernel,
      mesh=mesh,
      out_type=jax.ShapeDtypeStruct((250000, 128), jnp.float32),
      scratch_types=[
          pltpu.VMEM((32, 128), jnp.float32),
          pltpu.VMEM((32, 128), jnp.float32),
          pltpu.VMEM((32, 64), jnp.float32),
          pltpu.VMEM((16, 128), jnp.float32),
      ],
      compiler_params=pltpu.CompilerParams(use_tc_tiling_on_sc=True,
                                           needs_layout_passes=False),
  )
  def tk(tt_hbm, t2_hbm, gbuf, sbuf, gtail, stail):
    wid = lax.axis_index("s") * 2 + lax.axis_index("c")
    iota = lax.iota(jnp.int32, 16)
    rows01 = (iota, iota + 16)

    def blk(jj, carry):
      j = wid + _NW * jj

      @pl.when(j < _TB)
      def _():
        pltpu.sync_copy(tt_hbm.at[:, pl.ds(j * 128, 128)], gbuf)
        for r in range(32):
          for k in range(8):
            vals = plsc.load_gather(
                gbuf, [rows01[k % 2],
                       jnp.full((16,), 4 * r + k // 2, jnp.int32)])
            sbuf[r, pl.ds(16 * k, 16)] = vals
        pltpu.sync_copy(sbuf, t2_hbm.at[pl.ds(j * 32, 32), :])

      return carry

    lax.fori_loop(0, _NBLK, blk, 0)

    @pl.when(wid == 0)
    def _():
      pltpu.sync_copy(tt_hbm.at[:, pl.ds(_TB * 128, _TAIL)], gtail)
      for r in range(16):
        for k in range(8):
          vals = plsc.load_gather(
              gtail, [rows01[k % 2],
                      jnp.full((16,), 4 * r + k // 2, jnp.int32)])
          stail[r, pl.ds(16 * k, 16)] = vals
      pltpu.sync_copy(stail, t2_hbm.at[pl.ds(_TB * 32, 16), :])

  return tk


def _make_gather():
  mesh = plsc.VectorSubcoreMesh(core_axis_name="c", subcore_axis_name="s")

  @functools.partial(
      pl.kernel,
      mesh=mesh,
      out_type=jax.ShapeDtypeStruct((_H, 4, 128, 8, 128), jnp.float32),
      scratch_types=[
          pltpu.VMEM((_BPW,), jnp.int32),
          pltpu.VMEM((_BPW, _D), jnp.float32),
          pltpu.VMEM((4, 4, 8, 128), jnp.float32),
          pltpu.SemaphoreType.DMA,
      ],
      compiler_params=pltpu.CompilerParams(use_tc_tiling_on_sc=False,
                                           needs_layout_passes=False),
  )
  def gk(t_hbm, idx_hbm, out_hbm, idx_v, rows_v, stg, sem):
    wid = lax.axis_index("s") * 2 + lax.axis_index("c")
    b0 = wid * _BPW
    iota = lax.iota(jnp.int32, 16)

    def hloop(h, carry):
      pltpu.sync_copy(idx_hbm.at[pl.ds(h * _B + b0, _BPW)], idx_v)
      pltpu.async_copy(t_hbm.at[idx_v], rows_v, sem).wait()

      def kloop(k, c2):
        base = 16 * k
        for jp in range(4):
          ridx = 128 * jp + base + iota
          for i in range(4):
            for f in range(8):
              vals = plsc.load_gather(
                  rows_v, [ridx, jnp.full((16,), 8 * i + f, jnp.int32)])
              stg[i, jp, f, pl.ds(base, 16)] = vals
        return c2

      lax.fori_loop(0, 8, kloop, 0)
      for i in range(4):
        pltpu.sync_copy(stg.at[i], out_hbm.at[h, i, pl.ds(4 * wid, 4)])
      return carry

    lax.fori_loop(0, _H, hloop, 0)

  return gk


_transpose = _make_transpose()
_gather = _make_gather()


def kernel(x, table):
  t2 = _transpose(table.T)
  t_lin = t2.reshape(_V, _D)
  idxT = x.T.reshape(_B * _H)
  out5 = _gather(t_lin, idxT)
  return out5.transpose(2, 4, 0, 1, 3).reshape(_B, _H, _D)
